# Initial kernel scaffold; baseline (speedup 1.0000x reference)
import jax
import jax.numpy as jnp
from jax.experimental import pallas as pl


def kernel(y_preds, y_trues):
    raise NotImplementedError("write your pallas kernel here")


# speedup vs baseline: 8.8981x; 8.8981x over previous
import functools
import jax
import jax.numpy as jnp
from jax.experimental import pallas as pl


_S = 512
_K = 64
_ROWS_PER_SLICE = 32


def _row_loss_kernel(p_ref, xr_ref, yr_ref, yc_ref, out_ref):
    i = pl.program_id(0)

    @pl.when(i % _ROWS_PER_SLICE == 0)
    def _init():
        out_ref[...] = jnp.zeros_like(out_ref)

    p = p_ref[0]
    x_r = xr_ref[0]
    y_r = yr_ref[0]
    y_c = yc_ref[0]

    valid = ((x_r > 0) & (y_r > 0)).astype(jnp.float32)

    iu = jax.lax.broadcasted_iota(jnp.int32, (_S, _K), 0)
    ox_t = jnp.where(iu == x_r, valid, 0.0)
    iv = jax.lax.broadcasted_iota(jnp.int32, (_K, _S), 1)
    oy = (iv == y_c).astype(jnp.float32)
    counts = jax.lax.dot_general(
        ox_t, oy, (((1,), (0,)), ((), ())),
        preferred_element_type=jnp.float32)
    true_mask = counts > 0.0

    e = jnp.exp(p)
    s_all = jnp.sum(e)
    s_true_p = jnp.sum(jnp.where(true_mask, e, 0.0))
    s_true_n = jnp.sum(jnp.where(true_mask, 1.0 / e, 0.0))

    neg = jnp.log(1.0 + jnp.maximum(s_all - s_true_p, 0.0))
    pos = jnp.log(1.0 + s_true_n)
    loss = (neg + pos) * (1.0 / _ROWS_PER_SLICE)

    out_ref[...] += jnp.full((1, 1, 128), loss, jnp.float32)


@jax.jit
def kernel(y_preds, y_trues):
    n_rows = y_preds.shape[0] * y_preds.shape[1] * y_preds.shape[2]
    p = y_preds.reshape(n_rows, _S, _S)
    yt = y_trues.astype(jnp.int32).reshape(n_rows, _K, 2)
    x_r = yt[:, :, 0].reshape(n_rows, 1, _K)
    y_r = yt[:, :, 1].reshape(n_rows, 1, _K)
    y_c = yt[:, :, 1].reshape(n_rows, _K, 1)

    out = pl.pallas_call(
        _row_loss_kernel,
        grid=(n_rows,),
        in_specs=[
            pl.BlockSpec((1, _S, _S), lambda i: (i, 0, 0)),
            pl.BlockSpec((1, 1, _K), lambda i: (i, 0, 0)),
            pl.BlockSpec((1, 1, _K), lambda i: (i, 0, 0)),
            pl.BlockSpec((1, _K, 1), lambda i: (i, 0, 0)),
        ],
        out_specs=pl.BlockSpec((1, 1, 128),
                               lambda i: (i // _ROWS_PER_SLICE, 0, 0)),
        out_shape=jax.ShapeDtypeStruct((n_rows // _ROWS_PER_SLICE, 1, 128),
                                       jnp.float32),
    )(p, x_r, y_r, y_c)

    losses = out[:, 0, 0]
    loss = jnp.mean(losses)
    return (loss, losses[0], losses[1], losses[2], losses[3], losses[4])
